# baseline (speedup 1.0000x reference)
import functools

import jax
import jax.numpy as jnp
from jax import lax
from jax.experimental import pallas as pl
from jax.experimental.pallas import tpu as pltpu
from jax.experimental.pallas import tpu_sc as plsc

B, S, D = 2, 2048, 1024
E, K = 8, 2
N = B * S
NH = N // 2
CAPH = NH
RH = E * CAPH
TILE = 512
TH_MAX = NH * K // TILE + E
TT = 256
NSTEP = NH // TT


def _make_gating_body(half):
    def body(x_ref, wg_ref, wexp_ref, wbf_prev_ref,
             pos1_ref, pos2_ref, w1_ref, w2_ref, et_ref, rt_ref, wbf_ref,
             carry_ref):
        del wbf_prev_ref
        step = pl.program_id(0)
        wbf_ref[...] = wexp_ref[...].astype(jnp.bfloat16)

        @pl.when(step == 0)
        def _init():
            carry_ref[...] = jnp.zeros((E, TT), jnp.float32)

        lg = lax.dot_general(wg_ref[...], x_ref[...],
                             (((1,), (1,)), ((), ())))
        e_iota = lax.broadcasted_iota(jnp.int32, (E, TT), 0)

        m1 = jnp.max(lg, axis=0, keepdims=True)
        i1 = jnp.min(jnp.where(lg == m1, e_iota, E), axis=0, keepdims=True)
        lg2 = jnp.where(e_iota == i1, -3.0e38, lg)
        m2 = jnp.max(lg2, axis=0, keepdims=True)
        i2 = jnp.min(jnp.where(lg2 == m2, e_iota, E), axis=0, keepdims=True)

        s = jnp.exp(m2 - m1)
        w1 = 1.0 / (1.0 + s)
        w2 = s * w1

        oh = ((e_iota == i1) | (e_iota == i2)).astype(jnp.float32)
        t_iota_r = lax.broadcasted_iota(jnp.int32, (TT, TT), 0)
        t_iota_c = lax.broadcasted_iota(jnp.int32, (TT, TT), 1)
        ustrict = (t_iota_r < t_iota_c).astype(jnp.float32)
        excl = lax.dot_general(oh, ustrict, (((1,), (0,)), ((), ())))

        carry = carry_ref[...]
        ranks = excl + carry
        rank1 = jnp.sum(jnp.where(e_iota == i1, ranks, 0.0), axis=0,
                        keepdims=True)
        rank2 = jnp.sum(jnp.where(e_iota == i2, ranks, 0.0), axis=0,
                        keepdims=True)
        pos1 = (i1 * CAPH + rank1.astype(jnp.int32))
        pos2 = (i2 * CAPH + rank2.astype(jnp.int32))

        pos1_ref[...] = pos1.reshape(1, 1, TT)
        pos2_ref[...] = pos2.reshape(1, 1, TT)
        w1_ref[...] = w1.reshape(1, 1, TT)
        w2_ref[...] = w2.reshape(1, 1, TT)

        tot = jnp.sum(oh, axis=1, keepdims=True)
        carry_new = carry + jnp.broadcast_to(tot, (E, TT))
        carry_ref[...] = carry_new

        @pl.when(step == NSTEP - 1)
        def _emit_tile_map():
            nt = jnp.floor((carry_new + (TILE - 1)) * (1.0 / TILE))
            tri = (lax.broadcasted_iota(jnp.int32, (E, E), 0)
                   >= lax.broadcasted_iota(jnp.int32, (E, E), 1)).astype(
                       jnp.float32)
            cum = lax.dot_general(tri, nt, (((1,), (0,)), ((), ())))
            j_lane = lax.broadcasted_iota(jnp.int32, (E, TT), 1).astype(
                jnp.float32)
            ge = (cum <= j_lane).astype(jnp.float32)
            e_t = jnp.sum(ge, axis=0, keepdims=True)
            start = cum - nt
            e_t_b = jnp.broadcast_to(e_t, (E, TT))
            e_iota_f = e_iota.astype(jnp.float32)
            st_g = jnp.sum(jnp.where(e_iota_f == e_t_b, start, 0.0), axis=0,
                           keepdims=True)
            j1 = lax.broadcasted_iota(jnp.int32, (1, TT), 1).astype(
                jnp.float32)
            total = cum[E - 1:E, :]
            valid = j1 < total
            et_ref[...] = jnp.where(valid, e_t, 0.0).astype(jnp.int32)
            rt_ref[...] = jnp.where(valid, j1 - st_g, 0.0).astype(jnp.int32)

    return body


def _gating(x2d, wg, wexp, wbf_prev, half):
    wmap = lambda i: (half * 4 + i // 2, i % 2, 0)
    kwargs = {}
    if wbf_prev is not None:
        kwargs["input_output_aliases"] = {3: 6}
    return pl.pallas_call(
        _make_gating_body(half),
        grid=(NSTEP,),
        in_specs=[
            pl.BlockSpec((TT, D), lambda i, h=half: (h * NSTEP + i, 0)),
            pl.BlockSpec((E, D), lambda i: (0, 0)),
            pl.BlockSpec((1, D // 2, D), wmap),
            pl.BlockSpec(memory_space=pl.ANY),
        ],
        out_specs=[
            pl.BlockSpec((1, 1, TT), lambda i: (i, 0, 0)),
            pl.BlockSpec((1, 1, TT), lambda i: (i, 0, 0)),
            pl.BlockSpec((1, 1, TT), lambda i: (i, 0, 0)),
            pl.BlockSpec((1, 1, TT), lambda i: (i, 0, 0)),
            pl.BlockSpec((1, TT), lambda i: (0, 0)),
            pl.BlockSpec((1, TT), lambda i: (0, 0)),
            pl.BlockSpec((1, D // 2, D), wmap),
        ],
        out_shape=[
            jax.ShapeDtypeStruct((NSTEP, 1, TT), jnp.int32),
            jax.ShapeDtypeStruct((NSTEP, 1, TT), jnp.int32),
            jax.ShapeDtypeStruct((NSTEP, 1, TT), jnp.float32),
            jax.ShapeDtypeStruct((NSTEP, 1, TT), jnp.float32),
            jax.ShapeDtypeStruct((1, TT), jnp.int32),
            jax.ShapeDtypeStruct((1, TT), jnp.int32),
            jax.ShapeDtypeStruct((E, D, D), jnp.bfloat16),
        ],
        scratch_shapes=[pltpu.VMEM((E, TT), jnp.float32)],
        **kwargs,
    )(x2d, wg, wexp, wbf_prev if wbf_prev is not None else wexp)


def _make_scatter_kernel(half):
    def body(pos1_hbm, pos2_hbm, x_hbm, xs_hbm,
             idx1_v, idx2_v, xba, xbb,
             s1a, s2a, s1b, s2b, sla, slb):
        wid = lax.axis_index("s") * 2 + lax.axis_index("c")
        base = half * NH + wid * 64
        pltpu.sync_copy(pos1_hbm.at[pl.ds(wid * 2, 2)], idx1_v)
        pltpu.sync_copy(pos2_hbm.at[pl.ds(wid * 2, 2)], idx2_v)

        bufs = [(xba, s1a, s2a, sla), (xbb, s1b, s2b, slb)]

        def load(c):
            xbuf, _, _, sl = bufs[c % 2]
            h = pltpu.make_async_copy(
                x_hbm.at[pl.ds(base + c * 32, 32)], xbuf, sl)
            h.start()
            return h

        lh = {0: load(0)}
        scat_pending = [None, None]
        for c in range(2):
            if c < 1:
                lh[c + 1] = load(c + 1)
            lh[c].wait()
            xbuf, s1, s2, _ = bufs[c % 2]
            h1 = pltpu.make_async_copy(xbuf, xs_hbm.at[idx1_v.at[c]], s1)
            h2 = pltpu.make_async_copy(xbuf, xs_hbm.at[idx2_v.at[c]], s2)
            h1.start()
            h2.start()
            scat_pending[c % 2] = (h1, h2)
        for p in scat_pending:
            if p is not None:
                p[0].wait()
                p[1].wait()

    return body


def _scatter_x(pos1_2d, pos2_2d, x2d, half):
    mesh = plsc.VectorSubcoreMesh(core_axis_name="c", subcore_axis_name="s")
    fn = functools.partial(
        pl.kernel,
        mesh=mesh,
        out_type=jax.ShapeDtypeStruct((RH, D), jnp.float32),
        scratch_types=[
            pltpu.VMEM((2, 32), jnp.int32),
            pltpu.VMEM((2, 32), jnp.int32),
            pltpu.VMEM((32, D), jnp.float32),
            pltpu.VMEM((32, D), jnp.float32),
            pltpu.SemaphoreType.DMA,
            pltpu.SemaphoreType.DMA,
            pltpu.SemaphoreType.DMA,
            pltpu.SemaphoreType.DMA,
            pltpu.SemaphoreType.DMA,
            pltpu.SemaphoreType.DMA,
        ],
    )(_make_scatter_kernel(half))
    return fn(pos1_2d, pos2_2d, x2d)


def _mm_body(et_ref, rt_ref, x_ref, w_ref, y_ref):
    xb = x_ref[...].astype(jnp.bfloat16)
    y_ref[...] = lax.dot_general(
        xb, w_ref[0], (((1,), (1,)), ((), ())),
        preferred_element_type=jnp.float32)


def _grouped_matmul(xs, wbf, e_t, r_t):
    grid_spec = pltpu.PrefetchScalarGridSpec(
        num_scalar_prefetch=2,
        grid=(TH_MAX,),
        in_specs=[
            pl.BlockSpec((TILE, D),
                         lambda j, et, rt: (et[0, j] * (CAPH // TILE)
                                            + rt[0, j], 0)),
            pl.BlockSpec((1, D, D), lambda j, et, rt: (et[0, j], 0, 0)),
        ],
        out_specs=pl.BlockSpec(
            (TILE, D), lambda j, et, rt: (et[0, j] * (CAPH // TILE)
                                          + rt[0, j], 0)),
    )
    return pl.pallas_call(
        _mm_body,
        grid_spec=grid_spec,
        out_shape=jax.ShapeDtypeStruct((RH, D), jnp.float32),
    )(e_t, r_t, xs, wbf)


def _make_combine_kernel(half):
    def body(pos1_hbm, pos2_hbm, w1_hbm, w2_hbm, y_hbm, out_hbm,
             idx1_v, idx2_v, w1_v, w2_v,
             y1a, y2a, oa, y1b, y2b, ob,
             sem1a, sem2a, semoa, sem1b, sem2b, semob):
        wid = lax.axis_index("s") * 2 + lax.axis_index("c")
        base = wid * 64
        pltpu.sync_copy(pos1_hbm.at[pl.ds(wid * 4, 4)], idx1_v)
        pltpu.sync_copy(pos2_hbm.at[pl.ds(wid * 4, 4)], idx2_v)
        pltpu.sync_copy(w1_hbm.at[pl.ds(wid * 64, 64)], w1_v)
        pltpu.sync_copy(w2_hbm.at[pl.ds(wid * 64, 64)], w2_v)

        bufs = [(y1a, y2a, oa, sem1a, sem2a, semoa),
                (y1b, y2b, ob, sem1b, sem2b, semob)]

        def gathers(c):
            y1buf, y2buf, _, s1, s2, _ = bufs[c % 2]
            h1 = pltpu.make_async_copy(y_hbm.at[idx1_v.at[c]], y1buf, s1)
            h2 = pltpu.make_async_copy(y_hbm.at[idx2_v.at[c]], y2buf, s2)
            h1.start()
            h2.start()
            return h1, h2

        dnums = lax.GatherDimensionNumbers(
            offset_dims=(), collapsed_slice_dims=(0,), start_index_map=(0,))

        hs = {0: gathers(0)}
        out_pending = [None, None]
        for c in range(4):
            if c < 3:
                hs[c + 1] = gathers(c + 1)
            hs[c][0].wait()
            hs[c][1].wait()
            y1buf, y2buf, obuf, _, _, so = bufs[c % 2]
            if out_pending[c % 2] is not None:
                out_pending[c % 2].wait()
            w1blk = w1_v[pl.ds(c * 16, 16)]
            w2blk = w2_v[pl.ds(c * 16, 16)]

            def row_body(r, _):
                lane = jnp.full((16, 1), r, jnp.int32)
                w1s = lax.gather(
                    w1blk, lane, dnums, (1,),
                    mode=lax.GatherScatterMode.PROMISE_IN_BOUNDS)
                w2s = lax.gather(
                    w2blk, lane, dnums, (1,),
                    mode=lax.GatherScatterMode.PROMISE_IN_BOUNDS)
                for f in range(D // 16):
                    sl = pl.ds(f * 16, 16)
                    obuf[r, sl] = y1buf[r, sl] * w1s + y2buf[r, sl] * w2s
                return 0

            lax.fori_loop(0, 16, row_body, 0)
            oh = pltpu.make_async_copy(
                obuf, out_hbm.at[pl.ds(base + c * 16, 16)], so)
            oh.start()
            out_pending[c % 2] = oh
        out_pending[0].wait()
        out_pending[1].wait()

    return body


def _combine(pos1_e, pos2_e, w1, w2, y, half):
    mesh = plsc.VectorSubcoreMesh(core_axis_name="c", subcore_axis_name="s")
    fn = functools.partial(
        pl.kernel,
        mesh=mesh,
        out_type=jax.ShapeDtypeStruct((NH, D), jnp.float32),
        scratch_types=[
            pltpu.VMEM((4, 16), jnp.int32),
            pltpu.VMEM((4, 16), jnp.int32),
            pltpu.VMEM((64,), jnp.float32),
            pltpu.VMEM((64,), jnp.float32),
            pltpu.VMEM((16, D), jnp.float32),
            pltpu.VMEM((16, D), jnp.float32),
            pltpu.VMEM((16, D), jnp.float32),
            pltpu.VMEM((16, D), jnp.float32),
            pltpu.VMEM((16, D), jnp.float32),
            pltpu.VMEM((16, D), jnp.float32),
            pltpu.SemaphoreType.DMA,
            pltpu.SemaphoreType.DMA,
            pltpu.SemaphoreType.DMA,
            pltpu.SemaphoreType.DMA,
            pltpu.SemaphoreType.DMA,
            pltpu.SemaphoreType.DMA,
        ],
    )(_make_combine_kernel(half))
    return fn(pos1_e, pos2_e, w1, w2, y)


def kernel(x, Wg, Wexp):
    x2d = x.reshape(N, D)

    halves = []
    wbf = None
    for half in range(2):
        p1, p2, w1, w2, e_t, r_t, wbf = _gating(x2d, Wg, Wexp, wbf, half)
        halves.append((p1, p2, w1.reshape(NH), w2.reshape(NH), e_t, r_t))

    outs = []
    for half in range(2):
        p1, p2, w1, w2, e_t, r_t = halves[half]
        xs = _scatter_x(p1.reshape(NH // 32, 32), p2.reshape(NH // 32, 32),
                        x2d, half)
        y = _grouped_matmul(xs, wbf, e_t, r_t)
        out_h = _combine(p1.reshape(NH // 16, 16), p2.reshape(NH // 16, 16),
                         w1, w2, y, half)
        outs.append(out_h)
    return jnp.concatenate(outs, axis=0).reshape(B, S, D)
